# baseline (speedup 1.0000x reference)
import jax
import jax.numpy as jnp
from jax.experimental import pallas as pl
from jax.experimental.pallas import tpu as pltpu

B = 512
D = 4096
E = 64
K = 8
IB = 64


def _router_kernel(x_ref, wt_ref, gates2_ref, mask_ref, ones_ref):
    x = x_ref[...]
    wt = wt_ref[...]
    logits = jnp.dot(x, wt, preferred_element_type=jnp.float32)
    m = jnp.max(logits, axis=-1, keepdims=True)
    ex = jnp.exp(logits - m)
    s = jnp.sum(ex, axis=-1, keepdims=True)
    gates = ex / s

    col = jax.lax.broadcasted_iota(jnp.int32, (B, E), 1)
    work = gates
    mask = jnp.zeros((B, E), jnp.float32)
    for _ in range(K):
        mx = jnp.max(work, axis=-1, keepdims=True)
        cand = jnp.where(work == mx, col, E)
        first = jnp.min(cand, axis=-1, keepdims=True)
        pick = col == first
        mask = jnp.where(pick, 1.0, mask)
        work = jnp.where(pick, -1.0, work)

    gates2_ref[:, 0:E] = gates
    gates2_ref[:, E:2 * E] = gates
    mask_ref[...] = mask
    ones_ref[...] = jnp.ones((B, E), jnp.float32)


def _combine_kernel(gates2_ref, mask2_ref, out_ref):
    g = gates2_ref[...]
    m2 = mask2_ref[...]
    out_ref[...] = g[:, None, :] * m2[None, :, :]


def kernel(x, W):
    wt = W.T
    gates2, mask, ones = pl.pallas_call(
        _router_kernel,
        out_shape=(
            jax.ShapeDtypeStruct((B, 2 * E), jnp.float32),
            jax.ShapeDtypeStruct((B, E), jnp.float32),
            jax.ShapeDtypeStruct((B, E), jnp.float32),
        ),
    )(x, wt)

    mask2 = mask.reshape(B // 2, 2 * E)
    out = pl.pallas_call(
        _combine_kernel,
        grid=(B // IB,),
        in_specs=[
            pl.BlockSpec((IB, 2 * E), lambda i: (i, 0)),
            pl.BlockSpec((B // 2, 2 * E), lambda i: (0, 0)),
        ],
        out_specs=pl.BlockSpec((IB, B // 2, 2 * E), lambda i: (i, 0, 0)),
        out_shape=jax.ShapeDtypeStruct((B, B // 2, 2 * E), jnp.float32),
    )(gates2, mask2)

    combine = out.reshape(B, B, E)
    return (combine, mask, ones)


# speedup vs baseline: 1.1124x; 1.1124x over previous
import jax
import jax.numpy as jnp
from jax.experimental import pallas as pl
from jax.experimental.pallas import tpu as pltpu

B = 512
D = 4096
E = 64
K = 8
IB = 64


def _fused_kernel(x_ref, wt_ref, out_ref, mask_ref, ones_ref,
                  gates_s, mask_s):
    i = pl.program_id(0)

    @pl.when(i == 0)
    def _router():
        x = x_ref[...]
        wt = wt_ref[...]
        logits = jnp.dot(x, wt, preferred_element_type=jnp.float32)
        m = jnp.max(logits, axis=-1, keepdims=True)
        ex = jnp.exp(logits - m)
        s = jnp.sum(ex, axis=-1, keepdims=True)
        gates = ex / s

        col = jax.lax.broadcasted_iota(jnp.int32, (B, E), 1)
        work = gates
        mask = jnp.zeros((B, E), jnp.float32)
        for _ in range(K):
            mx = jnp.max(work, axis=-1, keepdims=True)
            cand = jnp.where(work == mx, col, E)
            first = jnp.min(cand, axis=-1, keepdims=True)
            pick = col == first
            mask = jnp.where(pick, 1.0, mask)
            work = jnp.where(pick, -1.0, work)

        gates_s[...] = gates
        mask_s[...] = mask
        mask_ref[...] = mask
        ones_ref[...] = jnp.ones((B, E), jnp.float32)

    g = gates_s[pl.ds(i * IB, IB), :]
    m2 = mask_s[...]
    out_ref[...] = g[:, None, :] * m2[None, :, :]


def kernel(x, W):
    wt = W.T
    out, mask, ones = pl.pallas_call(
        _fused_kernel,
        grid=(B // IB,),
        in_specs=[
            pl.BlockSpec((B, D), lambda i: (0, 0)),
            pl.BlockSpec((D, E), lambda i: (0, 0)),
        ],
        out_specs=(
            pl.BlockSpec((IB, B, E), lambda i: (i, 0, 0)),
            pl.BlockSpec((B, E), lambda i: (0, 0)),
            pl.BlockSpec((B, E), lambda i: (0, 0)),
        ),
        out_shape=(
            jax.ShapeDtypeStruct((B, B, E), jnp.float32),
            jax.ShapeDtypeStruct((B, E), jnp.float32),
            jax.ShapeDtypeStruct((B, E), jnp.float32),
        ),
        scratch_shapes=[
            pltpu.VMEM((B, E), jnp.float32),
            pltpu.VMEM((B, E), jnp.float32),
        ],
    )(x, wt)
    return (out, mask, ones)


# speedup vs baseline: 4.3797x; 3.9373x over previous
import jax
import jax.numpy as jnp
from jax.experimental import pallas as pl
from jax.experimental.pallas import tpu as pltpu

B = 512
D = 4096
E = 64
K = 8
IB = 64


def _fused_kernel(x_ref, wt_ref, out_ref, mask_ref, ones_ref,
                  gatest_s, maskt_s):
    i = pl.program_id(0)

    @pl.when(i == 0)
    def _router():
        x = x_ref[...]
        wt = wt_ref[...]
        logits = jnp.dot(x, wt, preferred_element_type=jnp.float32)
        m = jnp.max(logits, axis=-1, keepdims=True)
        ex = jnp.exp(logits - m)
        s = jnp.sum(ex, axis=-1, keepdims=True)
        gates = ex / s

        col = jax.lax.broadcasted_iota(jnp.int32, (B, E), 1)
        work = gates
        mask = jnp.zeros((B, E), jnp.float32)
        for _ in range(K):
            mx = jnp.max(work, axis=-1, keepdims=True)
            cand = jnp.where(work == mx, col, E)
            first = jnp.min(cand, axis=-1, keepdims=True)
            pick = col == first
            mask = jnp.where(pick, 1.0, mask)
            work = jnp.where(pick, -1.0, work)

        gatest_s[...] = gates
        maskt_s[...] = jnp.transpose(mask)
        mask_ref[...] = mask
        ones_ref[...] = jnp.ones((B, E), jnp.float32)

    mt = maskt_s[...]
    g_blk = gatest_s[pl.ds(i * IB, IB), :]
    out_ref[...] = g_blk[:, :, None] * mt[None, :, :]


def kernel(x, W):
    wt = W.T
    outt, mask, ones = pl.pallas_call(
        _fused_kernel,
        grid=(B // IB,),
        in_specs=[
            pl.BlockSpec((B, D), lambda i: (0, 0)),
            pl.BlockSpec((D, E), lambda i: (0, 0)),
        ],
        out_specs=(
            pl.BlockSpec((IB, E, B), lambda i: (i, 0, 0)),
            pl.BlockSpec((B, E), lambda i: (0, 0)),
            pl.BlockSpec((B, E), lambda i: (0, 0)),
        ),
        out_shape=(
            jax.ShapeDtypeStruct((B, E, B), jnp.float32),
            jax.ShapeDtypeStruct((B, E), jnp.float32),
            jax.ShapeDtypeStruct((B, E), jnp.float32),
        ),
        scratch_shapes=[
            pltpu.VMEM((B, E), jnp.float32),
            pltpu.VMEM((E, B), jnp.float32),
        ],
    )(x, wt)
    combine = jnp.transpose(outt, (0, 2, 1))
    return (combine, mask, ones)


# speedup vs baseline: 4.5092x; 1.0296x over previous
import jax
import jax.numpy as jnp
from jax.experimental import pallas as pl
from jax.experimental.pallas import tpu as pltpu

B = 512
D = 4096
E = 64
K = 8
IB = 32


def _fused_kernel(x_ref, wt_ref, out_ref, mask_ref, ones_ref,
                  gatest_s, maskt_s):
    i = pl.program_id(0)

    @pl.when(i == 0)
    def _router():
        x = x_ref[...]
        wt = wt_ref[...]
        logits = jnp.dot(x, wt, preferred_element_type=jnp.float32)
        m = jnp.max(logits, axis=-1, keepdims=True)
        ex = jnp.exp(logits - m)
        s = jnp.sum(ex, axis=-1, keepdims=True)
        gates = ex / s

        col = jax.lax.broadcasted_iota(jnp.int32, (B, E), 1)
        work = gates
        mask = jnp.zeros((B, E), jnp.float32)
        for _ in range(K):
            mx = jnp.max(work, axis=-1, keepdims=True)
            cand = jnp.where(work == mx, col, E)
            first = jnp.min(cand, axis=-1, keepdims=True)
            pick = col == first
            mask = jnp.where(pick, 1.0, mask)
            work = jnp.where(pick, -1.0, work)

        gatest_s[...] = gates
        maskt_s[...] = jnp.transpose(mask)
        mask_ref[...] = mask
        ones_ref[...] = jnp.ones((B, E), jnp.float32)

    mt = maskt_s[...]
    g_blk = gatest_s[pl.ds(i * IB, IB), :]
    out_ref[...] = g_blk[:, :, None] * mt[None, :, :]


def kernel(x, W):
    wt = W.T
    outt, mask, ones = pl.pallas_call(
        _fused_kernel,
        grid=(B // IB,),
        in_specs=[
            pl.BlockSpec((B, D), lambda i: (0, 0)),
            pl.BlockSpec((D, E), lambda i: (0, 0)),
        ],
        out_specs=(
            pl.BlockSpec((IB, E, B), lambda i: (i, 0, 0)),
            pl.BlockSpec((B, E), lambda i: (0, 0)),
            pl.BlockSpec((B, E), lambda i: (0, 0)),
        ),
        out_shape=(
            jax.ShapeDtypeStruct((B, E, B), jnp.float32),
            jax.ShapeDtypeStruct((B, E), jnp.float32),
            jax.ShapeDtypeStruct((B, E), jnp.float32),
        ),
        scratch_shapes=[
            pltpu.VMEM((B, E), jnp.float32),
            pltpu.VMEM((E, B), jnp.float32),
        ],
    )(x, wt)
    combine = jnp.transpose(outt, (0, 2, 1))
    return (combine, mask, ones)
